# Initial kernel scaffold; baseline (speedup 1.0000x reference)
import jax
import jax.numpy as jnp
from jax.experimental import pallas as pl


def kernel(x, edge_index, edge_type, batch, embed, W1, b1, W2, b2, Wlin, blin):
    raise NotImplementedError("write your pallas kernel here")


# speedup vs baseline: 12.2975x; 12.2975x over previous
import functools

import jax
import jax.numpy as jnp
from jax import lax
from jax.experimental import pallas as pl
from jax.experimental.pallas import tpu as pltpu
from jax.experimental.pallas import tpu_sc as plsc

N = 50000
E = 800000
V = 10000
D = 64
G = 256

NC, NS = 2, 16
NP = 50176
EP = 819200
GP = 264
NB = NP // 256
HD = D // 2

_mesh = plsc.VectorSubcoreMesh(
    core_axis_name="c", subcore_axis_name="s", num_cores=NC, num_subcores=NS)


def _loop(n, f):
  lax.fori_loop(0, n, lambda i, c: (f(i), 0)[1], 0)


def _f32(*shape):
  return jax.ShapeDtypeStruct(shape, jnp.float32)


@functools.partial(
    pl.kernel,
    out_type=[_f32(NP, D), _f32(NC, NP), _f32(NC, GP)],
    mesh=_mesh,
    compiler_params=pltpu.CompilerParams(use_tc_tiling_on_sc=False),
    scratch_types=[
        pltpu.VMEM((14, 112), jnp.int32),
        pltpu.VMEM((112, D), jnp.float32),
        pltpu.VMEM((8, 128), jnp.int32),
        pltpu.VMEM((14, 112), jnp.int32),
        pltpu.VMEM((128,), jnp.float32),
        pltpu.VMEM_SHARED((NP,), jnp.float32),
        pltpu.VMEM_SHARED((GP,), jnp.float32),
        pltpu.SemaphoreType.DMA,
    ],
)
def _sc_embed_hist(x2, dst2, batch2, embed, z1,
                   h0_out, deg_out, cnt_out,
                   xbuf, rows, dbuf, bbuf, ones, deg_sh, cnt_sh, sem):
  cid = lax.axis_index("c")
  sid = lax.axis_index("s")
  wid = sid * NC + cid

  for i in range(8):
    ones[pl.ds(i * 16, 16)] = jnp.ones((16,), jnp.float32)
  pltpu.sync_copy(z1, deg_sh.at[pl.ds(sid * 3136, 3136)])

  @pl.when(sid == 0)
  def _():
    pltpu.sync_copy(z1.at[pl.ds(0, GP)], cnt_sh)

  plsc.subcore_barrier()

  pltpu.sync_copy(x2.at[pl.ds(wid * 14, 14), :], xbuf)

  def emb_step(j):
    pltpu.async_copy(embed.at[xbuf.at[j]], rows, sem).wait()
    pltpu.sync_copy(rows, h0_out.at[pl.ds(wid * 1568 + j * 112, 112), :])
  _loop(14, emb_step)

  dbase = cid * 3200 + sid * 200

  def deg_step(blk):
    pltpu.sync_copy(dst2.at[pl.ds(dbase + blk * 8, 8), :], dbuf)
    for r in range(8):
      pltpu.sync_copy(ones, deg_sh.at[dbuf.at[r]], add=True)
  _loop(25, deg_step)

  pltpu.sync_copy(batch2.at[pl.ds(cid * 224 + sid * 14, 14), :], bbuf)

  def cnt_step(j):
    pltpu.sync_copy(ones.at[pl.ds(0, 112)], cnt_sh.at[bbuf.at[j]], add=True)
  _loop(14, cnt_step)

  plsc.subcore_barrier()
  pltpu.sync_copy(deg_sh.at[pl.ds(sid * 3136, 3136)],
                  deg_out.at[cid, pl.ds(sid * 3136, 3136)])

  @pl.when(sid == 0)
  def _():
    pltpu.sync_copy(cnt_sh, cnt_out.at[cid])


@functools.partial(
    pl.kernel,
    out_type=_f32(2 * NP, HD),
    mesh=_mesh,
    compiler_params=pltpu.CompilerParams(use_tc_tiling_on_sc=False),
    scratch_types=[
        pltpu.VMEM((8, 128), jnp.int32),
        pltpu.VMEM((8, 128), jnp.int32),
        pltpu.VMEM((128, HD), jnp.float32),
        pltpu.VMEM((128, HD), jnp.float32),
        pltpu.VMEM_SHARED((NP, HD), jnp.float32),
        pltpu.SemaphoreType.DMA,
        pltpu.SemaphoreType.DMA,
    ],
)
def _sc_conv_scatter(u, src3, dst2, zS, S_out,
                     sbuf, dbuf, rows0, rows1, S_sh, sem0, sem1):
  cid = lax.axis_index("c")
  sid = lax.axis_index("s")

  pltpu.sync_copy(zS, S_sh.at[pl.ds(sid * 3136, 3136), :])
  plsc.subcore_barrier()

  rbase = sid * 400
  bufs = (rows0, rows1)
  sems = (sem0, sem1)

  def blk_step(blk):
    pltpu.sync_copy(src3.at[cid, pl.ds(rbase + blk * 8, 8), :], sbuf)
    pltpu.sync_copy(dst2.at[pl.ds(rbase + blk * 8, 8), :], dbuf)
    pltpu.async_copy(u.at[sbuf.at[0]], bufs[0], sems[0])
    for r in range(8):
      if r + 1 < 8:
        pltpu.async_copy(u.at[sbuf.at[r + 1]], bufs[(r + 1) % 2],
                         sems[(r + 1) % 2])
      pltpu.make_async_copy(u.at[sbuf.at[r]], bufs[r % 2], sems[r % 2]).wait()
      pltpu.sync_copy(bufs[r % 2], S_sh.at[dbuf.at[r]], add=True)
  _loop(50, blk_step)

  plsc.subcore_barrier()
  pltpu.sync_copy(S_sh.at[pl.ds(sid * 3136, 3136), :],
                  S_out.at[pl.ds(cid * NP + sid * 3136, 3136), :])


@functools.partial(
    pl.kernel,
    out_type=_f32(NC, GP, D),
    mesh=_mesh,
    compiler_params=pltpu.CompilerParams(use_tc_tiling_on_sc=False),
    scratch_types=[
        pltpu.VMEM((14, 112), jnp.int32),
        pltpu.VMEM((112, D), jnp.float32),
        pltpu.VMEM_SHARED((GP, D), jnp.float32),
    ],
)
def _sc_pool(h2, batch2, zG, pool_out, bbuf, hbuf, pool_sh):
  cid = lax.axis_index("c")
  sid = lax.axis_index("s")

  @pl.when(sid == 0)
  def _():
    pltpu.sync_copy(zG, pool_sh)

  plsc.subcore_barrier()

  base = cid * 25088 + sid * 1568
  pltpu.sync_copy(batch2.at[pl.ds(cid * 224 + sid * 14, 14), :], bbuf)

  def pool_step(j):
    pltpu.sync_copy(h2.at[pl.ds(base + j * 112, 112), :], hbuf)
    pltpu.sync_copy(hbuf, pool_sh.at[bbuf.at[j]], add=True)
  _loop(14, pool_step)

  plsc.subcore_barrier()

  @pl.when(sid == 0)
  def _():
    pltpu.sync_copy(pool_sh, pool_out.at[cid])


def _tc_u1_body(h0, deg, W1h, u1, dinv_out):
  dinv = lax.rsqrt(deg[...] + 1.0)
  u1[...] = jnp.dot(h0[...], W1h[0],
                    preferred_element_type=jnp.float32) * dinv
  dinv_out[...] = dinv


def _tc_u1(h0, deg_v, W1):
  return pl.pallas_call(
      _tc_u1_body,
      grid=(2 * NB,),
      in_specs=[
          pl.BlockSpec((256, D), lambda j: (lax.rem(j, NB), 0)),
          pl.BlockSpec((256, 1), lambda j: (lax.rem(j, NB), 0)),
          pl.BlockSpec((1, D, HD), lambda j: (j // NB, 0, 0)),
      ],
      out_specs=[
          pl.BlockSpec((256, HD), lambda j: (j, 0)),
          pl.BlockSpec((256, 1), lambda j: (lax.rem(j, NB), 0)),
      ],
      out_shape=[_f32(2 * NP, HD), _f32(NP, 1)],
  )(h0, deg_v, W1)


def _tc_u2_body(Sa, Sb, ua, ub, dinv, b1, W2h, u2):
  d = dinv[...]
  ha = jnp.maximum(d * (Sa[...] + ua[...]) + b1[...][:, :HD], 0.0)
  hb = jnp.maximum(d * (Sb[...] + ub[...]) + b1[...][:, HD:], 0.0)
  h1 = jnp.concatenate([ha, hb], axis=1)
  u2[...] = jnp.dot(h1, W2h[0], preferred_element_type=jnp.float32) * d


def _tc_u2(S1, u1, dinv, b1, W2):
  rmap = lambda j: (lax.rem(j, NB), 0)
  rmapb = lambda j: (lax.rem(j, NB) + NB, 0)
  return pl.pallas_call(
      _tc_u2_body,
      grid=(2 * NB,),
      in_specs=[
          pl.BlockSpec((256, HD), rmap),
          pl.BlockSpec((256, HD), rmapb),
          pl.BlockSpec((256, HD), rmap),
          pl.BlockSpec((256, HD), rmapb),
          pl.BlockSpec((256, 1), rmap),
          pl.BlockSpec((1, D), lambda j: (0, 0)),
          pl.BlockSpec((1, D, HD), lambda j: (j // NB, 0, 0)),
      ],
      out_specs=pl.BlockSpec((256, HD), lambda j: (j, 0)),
      out_shape=_f32(2 * NP, HD),
  )(S1, S1, u1, u1, dinv, b1, W2)


def _tc_h2_body(Sa, Sb, ua, ub, dinv, b2, h2):
  d = dinv[...]
  ha = jnp.maximum(d * (Sa[...] + ua[...]) + b2[...][:, :HD], 0.0)
  hb = jnp.maximum(d * (Sb[...] + ub[...]) + b2[...][:, HD:], 0.0)
  h2[...] = jnp.concatenate([ha, hb], axis=1)


def _tc_h2(S2, u2, dinv, b2):
  rmap = lambda j: (j, 0)
  rmapb = lambda j: (j + NB, 0)
  return pl.pallas_call(
      _tc_h2_body,
      grid=(NB,),
      in_specs=[
          pl.BlockSpec((256, HD), rmap),
          pl.BlockSpec((256, HD), rmapb),
          pl.BlockSpec((256, HD), rmap),
          pl.BlockSpec((256, HD), rmapb),
          pl.BlockSpec((256, 1), rmap),
          pl.BlockSpec((1, D), lambda j: (0, 0)),
      ],
      out_specs=pl.BlockSpec((256, D), rmap),
      out_shape=_f32(NP, D),
  )(S2, S2, u2, u2, dinv, b2)


def _tc_head_body(p0, p1, cnt, Wl, bl, out):
  g = (p0[...] + p1[...]) / jnp.maximum(cnt[...], 1.0)
  out[...] = jnp.dot(g, Wl[...], preferred_element_type=jnp.float32) + bl[...]


def _tc_head(p0, p1, cnt, Wl, bl):
  return pl.pallas_call(
      _tc_head_body,
      out_shape=_f32(G, 128),
  )(p0, p1, cnt, Wl, bl)


def kernel(x, edge_index, edge_type, batch, embed, W1, b1, W2, b2, Wlin, blin):
  del edge_type
  f32 = jnp.float32

  x2 = jnp.pad(x.astype(jnp.int32), (0, NP - N)).reshape(448, 112)
  src = jnp.pad(edge_index[0].astype(jnp.int32), (0, EP - E),
                constant_values=N).reshape(6400, 128)
  dst2 = jnp.pad(edge_index[1].astype(jnp.int32), (0, EP - E),
                 constant_values=N).reshape(6400, 128)
  src3 = jnp.stack([src, src + NP])
  batch2 = jnp.pad(batch.astype(jnp.int32), (0, NP - N),
                   constant_values=G).reshape(448, 112)
  z1 = jnp.zeros((3136,), f32)
  zS = jnp.zeros((3136, HD), f32)
  zG = jnp.zeros((GP, D), f32)
  b1r = b1.reshape(1, D)
  b2r = b2.reshape(1, D)
  W1s = jnp.stack([W1[:, :HD], W1[:, HD:]])
  W2s = jnp.stack([W2[:, :HD], W2[:, HD:]])
  Wl = jnp.zeros((D, 128), f32).at[:, :2].set(Wlin)
  bl = jnp.zeros((1, 128), f32).at[0, :2].set(blin)

  h0, deg2, cnt2 = _sc_embed_hist(x2, dst2, batch2, embed, z1)
  deg_v = (deg2[0] + deg2[1]).reshape(NP, 1)
  cnt = (cnt2[0, :G] + cnt2[1, :G]).reshape(G, 1)

  u1, dinv = _tc_u1(h0, deg_v, W1s)
  S1 = _sc_conv_scatter(u1, src3, dst2, zS)
  u2 = _tc_u2(S1, u1, dinv, b1r, W2s)

  S2 = _sc_conv_scatter(u2, src3, dst2, zS)
  h2 = _tc_h2(S2, u2, dinv, b2r)

  pool = _sc_pool(h2, batch2, zG)
  out = _tc_head(pool[0, :G], pool[1, :G], cnt, Wl, bl)
  return out[:, :2]


# speedup vs baseline: 18.3518x; 1.4923x over previous
import functools

import jax
import jax.numpy as jnp
from jax import lax
from jax.experimental import pallas as pl
from jax.experimental.pallas import tpu as pltpu
from jax.experimental.pallas import tpu_sc as plsc

N = 50000
E = 800000
V = 10000
D = 64
G = 256

NC, NS = 2, 16
NP = 50176
EP = 819200
GP = 264
NB = NP // 256
HD = D // 2

_mesh = plsc.VectorSubcoreMesh(
    core_axis_name="c", subcore_axis_name="s", num_cores=NC, num_subcores=NS)


def _loop(n, f):
  lax.fori_loop(0, n, lambda i, c: (f(i), 0)[1], 0)


def _f32(*shape):
  return jax.ShapeDtypeStruct(shape, jnp.float32)


@functools.partial(
    pl.kernel,
    out_type=[_f32(NP, D), _f32(NC, NP), _f32(NC, GP)],
    mesh=_mesh,
    compiler_params=pltpu.CompilerParams(use_tc_tiling_on_sc=False),
    scratch_types=[
        pltpu.VMEM((14, 112), jnp.int32),
        pltpu.VMEM((112, D), jnp.float32),
        pltpu.VMEM((8, 128), jnp.int32),
        pltpu.VMEM((14, 112), jnp.int32),
        pltpu.VMEM((128,), jnp.float32),
        pltpu.VMEM_SHARED((NP,), jnp.float32),
        pltpu.VMEM_SHARED((GP,), jnp.float32),
        pltpu.SemaphoreType.DMA,
    ],
)
def _sc_embed_hist(x2, dst2, batch2, embed, z1,
                   h0_out, deg_out, cnt_out,
                   xbuf, rows, dbuf, bbuf, ones, deg_sh, cnt_sh, sem):
  cid = lax.axis_index("c")
  sid = lax.axis_index("s")
  wid = sid * NC + cid

  for i in range(8):
    ones[pl.ds(i * 16, 16)] = jnp.ones((16,), jnp.float32)
  pltpu.sync_copy(z1, deg_sh.at[pl.ds(sid * 3136, 3136)])

  @pl.when(sid == 0)
  def _():
    pltpu.sync_copy(z1.at[pl.ds(0, GP)], cnt_sh)

  plsc.subcore_barrier()

  pltpu.sync_copy(x2.at[pl.ds(wid * 14, 14), :], xbuf)

  def emb_step(j):
    pltpu.async_copy(embed.at[xbuf.at[j]], rows, sem).wait()
    pltpu.sync_copy(rows, h0_out.at[pl.ds(wid * 1568 + j * 112, 112), :])
  _loop(14, emb_step)

  dbase = cid * 3200 + sid * 200

  def deg_step(blk):
    pltpu.sync_copy(dst2.at[pl.ds(dbase + blk * 8, 8), :], dbuf)
    for r in range(8):
      pltpu.sync_copy(ones, deg_sh.at[dbuf.at[r]], add=True)
  _loop(25, deg_step)

  pltpu.sync_copy(batch2.at[pl.ds(cid * 224 + sid * 14, 14), :], bbuf)

  def cnt_step(j):
    pltpu.sync_copy(ones.at[pl.ds(0, 112)], cnt_sh.at[bbuf.at[j]], add=True)
  _loop(14, cnt_step)

  plsc.subcore_barrier()
  pltpu.sync_copy(deg_sh.at[pl.ds(sid * 3136, 3136)],
                  deg_out.at[cid, pl.ds(sid * 3136, 3136)])

  @pl.when(sid == 0)
  def _():
    pltpu.sync_copy(cnt_sh, cnt_out.at[cid])


@functools.partial(
    pl.kernel,
    out_type=_f32(2 * NP, HD),
    mesh=_mesh,
    compiler_params=pltpu.CompilerParams(use_tc_tiling_on_sc=False),
    scratch_types=[
        pltpu.VMEM((2, 20, 128), jnp.int32),
        pltpu.VMEM((2, 20, 128), jnp.int32),
        pltpu.VMEM((4, 128, HD), jnp.float32),
        pltpu.VMEM_SHARED((NP, HD), jnp.float32),
        pltpu.SemaphoreType.DMA,
        [pltpu.SemaphoreType.DMA] * 4,
        [pltpu.SemaphoreType.DMA] * 4,
    ],
)
def _sc_conv_scatter(u, src3, dst2, S_out,
                     sbuf, dbuf, rows, S_sh, isem, gs, ss):
  cid = lax.axis_index("c")
  sid = lax.axis_index("s")

  pltpu.sync_copy(u.at[pl.ds(cid * NP + sid * 3136, 3136), :],
                  S_sh.at[pl.ds(sid * 3136, 3136), :])
  plsc.subcore_barrier()

  def idx_fire(blk, par):
    base = sid * 400 + blk * 20
    pltpu.async_copy(src3.at[cid, pl.ds(base, 20), :], sbuf.at[par], isem)
    pltpu.async_copy(dst2.at[pl.ds(base, 20), :], dbuf.at[par], isem)

  def idx_wait(blk, par):
    base = sid * 400 + blk * 20
    pltpu.make_async_copy(
        src3.at[cid, pl.ds(base, 20), :], sbuf.at[par], isem).wait()
    pltpu.make_async_copy(
        dst2.at[pl.ds(base, 20), :], dbuf.at[par], isem).wait()

  def gfire(g, slot):
    p2 = lax.rem(g // 20, 2)
    r2 = lax.rem(g, 20)
    pltpu.async_copy(u.at[sbuf.at[p2, r2]], rows.at[slot], gs[slot])

  def sdrain(slot):
    pltpu.make_async_copy(rows.at[slot], S_sh.at[dbuf.at[0, 0]],
                          ss[slot]).wait()

  idx_fire(0, 0)
  idx_wait(0, 0)
  gfire(0, 0)
  gfire(1, 1)

  def q_step(q):
    b = q // 5
    j = lax.rem(q, 5)
    p = lax.rem(b, 2)
    for c in range(4):
      g = 4 * q + c

      @pl.when(g >= 2)
      def _():
        sdrain((c - 2) % 4)

      @pl.when(g + 2 < 400)
      def _():
        gfire(g + 2, (c + 2) % 4)
      row = 4 * j + c
      pltpu.make_async_copy(u.at[sbuf.at[p, row]], rows.at[c], gs[c]).wait()
      pltpu.async_copy(rows.at[c], S_sh.at[dbuf.at[p, row]], ss[c], add=True)

    @pl.when((j == 0) & (b + 1 < 20))
    def _():
      idx_fire(b + 1, 1 - p)

    @pl.when((j == 3) & (b + 1 < 20))
    def _():
      idx_wait(b + 1, 1 - p)
  _loop(100, q_step)

  sdrain(2)
  sdrain(3)

  plsc.subcore_barrier()
  pltpu.sync_copy(S_sh.at[pl.ds(sid * 3136, 3136), :],
                  S_out.at[pl.ds(cid * NP + sid * 3136, 3136), :])


@functools.partial(
    pl.kernel,
    out_type=_f32(NC, GP, D),
    mesh=_mesh,
    compiler_params=pltpu.CompilerParams(use_tc_tiling_on_sc=False),
    scratch_types=[
        pltpu.VMEM((14, 112), jnp.int32),
        pltpu.VMEM((112, D), jnp.float32),
        pltpu.VMEM_SHARED((GP, D), jnp.float32),
    ],
)
def _sc_pool(h2, batch2, zG, pool_out, bbuf, hbuf, pool_sh):
  cid = lax.axis_index("c")
  sid = lax.axis_index("s")

  @pl.when(sid == 0)
  def _():
    pltpu.sync_copy(zG, pool_sh)

  plsc.subcore_barrier()

  base = cid * 25088 + sid * 1568
  pltpu.sync_copy(batch2.at[pl.ds(cid * 224 + sid * 14, 14), :], bbuf)

  def pool_step(j):
    pltpu.sync_copy(h2.at[pl.ds(base + j * 112, 112), :], hbuf)
    pltpu.sync_copy(hbuf, pool_sh.at[bbuf.at[j]], add=True)
  _loop(14, pool_step)

  plsc.subcore_barrier()

  @pl.when(sid == 0)
  def _():
    pltpu.sync_copy(pool_sh, pool_out.at[cid])


RB = 1024
NJ = NP // RB
DL = RB // 256


def _tc_u1_body(h0, deg, W1h, u1, dinv_out):
  dcol = lax.rsqrt(deg[...] + 1.0)
  dinv_out[...] = dcol
  u1[...] = jnp.dot(h0[...], W1h[0],
                    preferred_element_type=jnp.float32) * dcol


def _tc_u1(h0, deg_l, W1):
  return pl.pallas_call(
      _tc_u1_body,
      grid=(2 * NJ,),
      in_specs=[
          pl.BlockSpec((RB, D), lambda j: (lax.rem(j, NJ), 0)),
          pl.BlockSpec((RB, 1), lambda j: (lax.rem(j, NJ), 0)),
          pl.BlockSpec((1, D, HD), lambda j: (j // NJ, 0, 0)),
      ],
      out_specs=[
          pl.BlockSpec((RB, HD), lambda j: (j, 0)),
          pl.BlockSpec((RB, 1), lambda j: (lax.rem(j, NJ), 0)),
      ],
      out_shape=[_f32(2 * NP, HD), _f32(NP, 1)],
  )(h0, deg_l, W1)


def _tc_u2_body(Sa, Sb, dinv, b1, W2h, u2):
  dcol = dinv[...]
  ha = jnp.maximum(dcol * Sa[...] + b1[...][:, :HD], 0.0)
  hb = jnp.maximum(dcol * Sb[...] + b1[...][:, HD:], 0.0)
  h1 = jnp.concatenate([ha, hb], axis=1)
  u2[...] = jnp.dot(h1, W2h[0], preferred_element_type=jnp.float32) * dcol


def _tc_u2(S1, dinv, b1, W2):
  rmap = lambda j: (lax.rem(j, NJ), 0)
  rmapb = lambda j: (lax.rem(j, NJ) + NJ, 0)
  return pl.pallas_call(
      _tc_u2_body,
      grid=(2 * NJ,),
      in_specs=[
          pl.BlockSpec((RB, HD), rmap),
          pl.BlockSpec((RB, HD), rmapb),
          pl.BlockSpec((RB, 1), lambda j: (lax.rem(j, NJ), 0)),
          pl.BlockSpec((1, D), lambda j: (0, 0)),
          pl.BlockSpec((1, D, HD), lambda j: (j // NJ, 0, 0)),
      ],
      out_specs=pl.BlockSpec((RB, HD), lambda j: (j, 0)),
      out_shape=_f32(2 * NP, HD),
  )(S1, S1, dinv, b1, W2)


def _tc_h2_body(Sa, Sb, dinv, b2, h2):
  dcol = dinv[...]
  ha = jnp.maximum(dcol * Sa[...] + b2[...][:, :HD], 0.0)
  hb = jnp.maximum(dcol * Sb[...] + b2[...][:, HD:], 0.0)
  h2[...] = jnp.concatenate([ha, hb], axis=1)


def _tc_h2(S2, dinv, b2):
  rmap = lambda j: (j, 0)
  rmapb = lambda j: (j + NJ, 0)
  return pl.pallas_call(
      _tc_h2_body,
      grid=(NJ,),
      in_specs=[
          pl.BlockSpec((RB, HD), rmap),
          pl.BlockSpec((RB, HD), rmapb),
          pl.BlockSpec((RB, 1), lambda j: (j, 0)),
          pl.BlockSpec((1, D), lambda j: (0, 0)),
      ],
      out_specs=pl.BlockSpec((RB, D), rmap),
      out_shape=_f32(NP, D),
  )(S2, S2, dinv, b2)


def _tc_head_body(p0, p1, cnt, Wl, bl, out):
  g = (p0[...] + p1[...]) / jnp.maximum(cnt[...], 1.0)
  out[...] = jnp.dot(g, Wl[...], preferred_element_type=jnp.float32) + bl[...]


def _tc_head(p0, p1, cnt, Wl, bl):
  return pl.pallas_call(
      _tc_head_body,
      out_shape=_f32(G, 128),
  )(p0, p1, cnt, Wl, bl)


def kernel(x, edge_index, edge_type, batch, embed, W1, b1, W2, b2, Wlin, blin):
  del edge_type
  f32 = jnp.float32

  x2 = jnp.pad(x.astype(jnp.int32), (0, NP - N)).reshape(448, 112)
  src = jnp.pad(edge_index[0].astype(jnp.int32), (0, EP - E),
                constant_values=N).reshape(6400, 128)
  dst2 = jnp.pad(edge_index[1].astype(jnp.int32), (0, EP - E),
                 constant_values=N).reshape(6400, 128)
  src3 = jnp.stack([src, src + NP])
  batch2 = jnp.pad(batch.astype(jnp.int32), (0, NP - N),
                   constant_values=G).reshape(448, 112)
  z1 = jnp.zeros((3136,), f32)
  zG = jnp.zeros((GP, D), f32)
  b1r = b1.reshape(1, D)
  b2r = b2.reshape(1, D)
  W1s = jnp.stack([W1[:, :HD], W1[:, HD:]])
  W2s = jnp.stack([W2[:, :HD], W2[:, HD:]])
  Wl = jnp.zeros((D, 128), f32).at[:, :2].set(Wlin)
  bl = jnp.zeros((1, 128), f32).at[0, :2].set(blin)

  h0, deg2, cnt2 = _sc_embed_hist(x2, dst2, batch2, embed, z1)
  deg_v = (deg2[0] + deg2[1]).reshape(NP, 1)
  cnt = (cnt2[0, :G] + cnt2[1, :G]).reshape(G, 1)

  u1, dinv = _tc_u1(h0, deg_v, W1s)
  S1 = _sc_conv_scatter(u1, src3, dst2)
  u2 = _tc_u2(S1, dinv, b1r, W2s)

  S2 = _sc_conv_scatter(u2, src3, dst2)
  h2 = _tc_h2(S2, dinv, b2r)

  pool = _sc_pool(h2, batch2, zG)
  out = _tc_head(pool[0, :G], pool[1, :G], cnt, Wl, bl)
  return out[:, :2]


# speedup vs baseline: 19.2842x; 1.0508x over previous
import functools

import jax
import jax.numpy as jnp
from jax import lax
from jax.experimental import pallas as pl
from jax.experimental.pallas import tpu as pltpu
from jax.experimental.pallas import tpu_sc as plsc

N = 50000
E = 800000
V = 10000
D = 64
G = 256

NC, NS = 2, 16
NP = 50176
EP = 819200
GP = 264
HD = D // 2

_mesh = plsc.VectorSubcoreMesh(
    core_axis_name="c", subcore_axis_name="s", num_cores=NC, num_subcores=NS)


def _loop(n, f):
  lax.fori_loop(0, n, lambda i, c: (f(i), 0)[1], 0)


def _f32(*shape):
  return jax.ShapeDtypeStruct(shape, jnp.float32)


@functools.partial(
    pl.kernel,
    out_type=[_f32(NP, D), _f32(NC, NP), _f32(NC, GP)],
    mesh=_mesh,
    compiler_params=pltpu.CompilerParams(use_tc_tiling_on_sc=False),
    scratch_types=[
        pltpu.VMEM((14, 112), jnp.int32),
        pltpu.VMEM((112, D), jnp.float32),
        pltpu.VMEM((8, 128), jnp.int32),
        pltpu.VMEM((14, 112), jnp.int32),
        pltpu.VMEM((128,), jnp.float32),
        pltpu.VMEM_SHARED((NP,), jnp.float32),
        pltpu.VMEM_SHARED((GP,), jnp.float32),
        pltpu.SemaphoreType.DMA,
    ],
)
def _sc_embed_hist(x2, dst2, batch2, embed, z1,
                   h0_out, deg_out, cnt_out,
                   xbuf, rows, dbuf, bbuf, ones, deg_sh, cnt_sh, sem):
  cid = lax.axis_index("c")
  sid = lax.axis_index("s")
  wid = sid * NC + cid

  for i in range(8):
    ones[pl.ds(i * 16, 16)] = jnp.ones((16,), jnp.float32)
  pltpu.sync_copy(z1, deg_sh.at[pl.ds(sid * 3136, 3136)])

  @pl.when(sid == 0)
  def _():
    pltpu.sync_copy(z1.at[pl.ds(0, GP)], cnt_sh)

  plsc.subcore_barrier()

  pltpu.sync_copy(x2.at[pl.ds(wid * 14, 14), :], xbuf)

  def emb_step(j):
    pltpu.async_copy(embed.at[xbuf.at[j]], rows, sem).wait()
    pltpu.sync_copy(rows, h0_out.at[pl.ds(wid * 1568 + j * 112, 112), :])
  _loop(14, emb_step)

  dbase = cid * 3200 + sid * 200

  def deg_step(blk):
    pltpu.sync_copy(dst2.at[pl.ds(dbase + blk * 8, 8), :], dbuf)
    for r in range(8):
      pltpu.sync_copy(ones, deg_sh.at[dbuf.at[r]], add=True)
  _loop(25, deg_step)

  pltpu.sync_copy(batch2.at[pl.ds(cid * 224 + sid * 14, 14), :], bbuf)

  def cnt_step(j):
    pltpu.sync_copy(ones.at[pl.ds(0, 112)], cnt_sh.at[bbuf.at[j]], add=True)
  _loop(14, cnt_step)

  plsc.subcore_barrier()
  pltpu.sync_copy(deg_sh.at[pl.ds(sid * 3136, 3136)],
                  deg_out.at[cid, pl.ds(sid * 3136, 3136)])

  @pl.when(sid == 0)
  def _():
    pltpu.sync_copy(cnt_sh, cnt_out.at[cid])


@functools.partial(
    pl.kernel,
    out_type=_f32(2 * NP, HD),
    mesh=_mesh,
    compiler_params=pltpu.CompilerParams(use_tc_tiling_on_sc=False),
    scratch_types=[
        pltpu.VMEM((2, 20, 128), jnp.int32),
        pltpu.VMEM((2, 20, 128), jnp.int32),
        pltpu.VMEM((4, 128, HD), jnp.float32),
        pltpu.VMEM_SHARED((NP, HD), jnp.float32),
        pltpu.SemaphoreType.DMA,
        [pltpu.SemaphoreType.DMA] * 4,
        [pltpu.SemaphoreType.DMA] * 4,
    ],
)
def _sc_conv_scatter(u, src3, dst2, S_out,
                     sbuf, dbuf, rows, S_sh, isem, gs, ss):
  cid = lax.axis_index("c")
  sid = lax.axis_index("s")

  pltpu.sync_copy(u.at[pl.ds(cid * NP + sid * 3136, 3136), :],
                  S_sh.at[pl.ds(sid * 3136, 3136), :])
  plsc.subcore_barrier()

  def idx_fire(blk, par):
    base = sid * 400 + blk * 20
    pltpu.async_copy(src3.at[cid, pl.ds(base, 20), :], sbuf.at[par], isem)
    pltpu.async_copy(dst2.at[pl.ds(base, 20), :], dbuf.at[par], isem)

  def idx_wait(blk, par):
    base = sid * 400 + blk * 20
    pltpu.make_async_copy(
        src3.at[cid, pl.ds(base, 20), :], sbuf.at[par], isem).wait()
    pltpu.make_async_copy(
        dst2.at[pl.ds(base, 20), :], dbuf.at[par], isem).wait()

  def gfire(g, slot):
    p2 = lax.rem(g // 20, 2)
    r2 = lax.rem(g, 20)
    pltpu.async_copy(u.at[sbuf.at[p2, r2]], rows.at[slot], gs[slot])

  def sdrain(slot):
    pltpu.make_async_copy(rows.at[slot], S_sh.at[dbuf.at[0, 0]],
                          ss[slot]).wait()

  idx_fire(0, 0)
  idx_wait(0, 0)
  gfire(0, 0)
  gfire(1, 1)

  def q_step(q):
    b = q // 5
    j = lax.rem(q, 5)
    p = lax.rem(b, 2)
    for c in range(4):
      g = 4 * q + c

      @pl.when(g >= 2)
      def _():
        sdrain((c - 2) % 4)

      @pl.when(g + 2 < 400)
      def _():
        gfire(g + 2, (c + 2) % 4)
      row = 4 * j + c
      pltpu.make_async_copy(u.at[sbuf.at[p, row]], rows.at[c], gs[c]).wait()
      pltpu.async_copy(rows.at[c], S_sh.at[dbuf.at[p, row]], ss[c], add=True)

    @pl.when((j == 0) & (b + 1 < 20))
    def _():
      idx_fire(b + 1, 1 - p)

    @pl.when((j == 3) & (b + 1 < 20))
    def _():
      idx_wait(b + 1, 1 - p)
  _loop(100, q_step)

  sdrain(2)
  sdrain(3)

  plsc.subcore_barrier()
  pltpu.sync_copy(S_sh.at[pl.ds(sid * 3136, 3136), :],
                  S_out.at[pl.ds(cid * NP + sid * 3136, 3136), :])


@functools.partial(
    pl.kernel,
    out_type=_f32(NC, GP, D),
    mesh=_mesh,
    compiler_params=pltpu.CompilerParams(use_tc_tiling_on_sc=False),
    scratch_types=[
        pltpu.VMEM((14, 112), jnp.int32),
        pltpu.VMEM((112, D), jnp.float32),
        pltpu.VMEM_SHARED((GP, D), jnp.float32),
    ],
)
def _sc_pool(h2, batch2, zG, pool_out, bbuf, hbuf, pool_sh):
  cid = lax.axis_index("c")
  sid = lax.axis_index("s")

  @pl.when(sid == 0)
  def _():
    pltpu.sync_copy(zG, pool_sh)

  plsc.subcore_barrier()

  base = cid * 25088 + sid * 1568
  pltpu.sync_copy(batch2.at[pl.ds(cid * 224 + sid * 14, 14), :], bbuf)

  def pool_step(j):
    pltpu.sync_copy(h2.at[pl.ds(base + j * 112, 112), :], hbuf)
    pltpu.sync_copy(hbuf, pool_sh.at[bbuf.at[j]], add=True)
  _loop(14, pool_step)

  plsc.subcore_barrier()

  @pl.when(sid == 0)
  def _():
    pltpu.sync_copy(pool_sh, pool_out.at[cid])


RB = 1792
NJ = NP // RB


def _tc_u1_body(h0, deg, W1h, u1, dinv_out):
  dcol = lax.rsqrt(deg[...] + 1.0)
  dinv_out[...] = dcol
  u1[...] = jnp.dot(h0[...], W1h[0],
                    preferred_element_type=jnp.float32) * dcol


def _tc_u1(h0, deg_v, W1):
  return pl.pallas_call(
      _tc_u1_body,
      grid=(2 * NJ,),
      in_specs=[
          pl.BlockSpec((RB, D), lambda j: (lax.rem(j, NJ), 0)),
          pl.BlockSpec((RB, 1), lambda j: (lax.rem(j, NJ), 0)),
          pl.BlockSpec((1, D, HD), lambda j: (j // NJ, 0, 0)),
      ],
      out_specs=[
          pl.BlockSpec((RB, HD), lambda j: (j, 0)),
          pl.BlockSpec((RB, 1), lambda j: (lax.rem(j, NJ), 0)),
      ],
      out_shape=[_f32(2 * NP, HD), _f32(NP, 1)],
  )(h0, deg_v, W1)


def _tc_u2_body(Sa, Sb, dinv, b1, W2h, u2):
  dcol = dinv[...]
  ha = jnp.maximum(dcol * Sa[...] + b1[...][:, :HD], 0.0)
  hb = jnp.maximum(dcol * Sb[...] + b1[...][:, HD:], 0.0)
  h1 = jnp.concatenate([ha, hb], axis=1)
  u2[...] = jnp.dot(h1, W2h[0], preferred_element_type=jnp.float32) * dcol


def _tc_u2(S1, dinv, b1, W2):
  rmap = lambda j: (lax.rem(j, NJ), 0)
  rmapb = lambda j: (lax.rem(j, NJ) + NJ, 0)
  return pl.pallas_call(
      _tc_u2_body,
      grid=(2 * NJ,),
      in_specs=[
          pl.BlockSpec((RB, HD), rmap),
          pl.BlockSpec((RB, HD), rmapb),
          pl.BlockSpec((RB, 1), rmap),
          pl.BlockSpec((1, D), lambda j: (0, 0)),
          pl.BlockSpec((1, D, HD), lambda j: (j // NJ, 0, 0)),
      ],
      out_specs=pl.BlockSpec((RB, HD), lambda j: (j, 0)),
      out_shape=_f32(2 * NP, HD),
  )(S1, S1, dinv, b1, W2)


def _tc_h2_body(Sa, Sb, dinv, b2, h2):
  dcol = dinv[...]
  ha = jnp.maximum(dcol * Sa[...] + b2[...][:, :HD], 0.0)
  hb = jnp.maximum(dcol * Sb[...] + b2[...][:, HD:], 0.0)
  h2[...] = jnp.concatenate([ha, hb], axis=1)


def _tc_h2(S2, dinv, b2):
  rmap = lambda j: (j, 0)
  rmapb = lambda j: (j + NJ, 0)
  return pl.pallas_call(
      _tc_h2_body,
      grid=(NJ,),
      in_specs=[
          pl.BlockSpec((RB, HD), rmap),
          pl.BlockSpec((RB, HD), rmapb),
          pl.BlockSpec((RB, 1), rmap),
          pl.BlockSpec((1, D), lambda j: (0, 0)),
      ],
      out_specs=pl.BlockSpec((RB, D), rmap),
      out_shape=_f32(NP, D),
  )(S2, S2, dinv, b2)


def _tc_head_body(p0, p1, cnt, Wl, bl, out):
  g = (p0[...] + p1[...]) / jnp.maximum(cnt[...], 1.0)
  out[...] = jnp.dot(g, Wl[...], preferred_element_type=jnp.float32) + bl[...]


def _tc_head(p0, p1, cnt, Wl, bl):
  return pl.pallas_call(
      _tc_head_body,
      out_shape=_f32(G, 128),
  )(p0, p1, cnt, Wl, bl)


def kernel(x, edge_index, edge_type, batch, embed, W1, b1, W2, b2, Wlin, blin):
  del edge_type
  f32 = jnp.float32

  x2 = jnp.pad(x.astype(jnp.int32), (0, NP - N)).reshape(448, 112)
  src = jnp.pad(edge_index[0].astype(jnp.int32), (0, EP - E),
                constant_values=N).reshape(6400, 128)
  dst2 = jnp.pad(edge_index[1].astype(jnp.int32), (0, EP - E),
                 constant_values=N).reshape(6400, 128)
  src3 = jnp.stack([src, src + NP])
  batch2 = jnp.pad(batch.astype(jnp.int32), (0, NP - N),
                   constant_values=G).reshape(448, 112)
  z1 = jnp.zeros((3136,), f32)
  zG = jnp.zeros((GP, D), f32)
  b1r = b1.reshape(1, D)
  b2r = b2.reshape(1, D)
  W1s = jnp.stack([W1[:, :HD], W1[:, HD:]])
  W2s = jnp.stack([W2[:, :HD], W2[:, HD:]])
  Wl = jnp.zeros((D, 128), f32).at[:, :2].set(Wlin)
  bl = jnp.zeros((1, 128), f32).at[0, :2].set(blin)

  h0, deg2, cnt2 = _sc_embed_hist(x2, dst2, batch2, embed, z1)
  deg_v = (deg2[0] + deg2[1]).reshape(NP, 1)
  cnt = (cnt2[0, :G] + cnt2[1, :G]).reshape(G, 1)

  u1, dinv = _tc_u1(h0, deg_v, W1s)
  S1 = _sc_conv_scatter(u1, src3, dst2)
  u2 = _tc_u2(S1, dinv, b1r, W2s)

  S2 = _sc_conv_scatter(u2, src3, dst2)
  h2 = _tc_h2(S2, dinv, b2r)

  pool = _sc_pool(h2, batch2, zG)
  out = _tc_head(pool[0, :G], pool[1, :G], cnt, Wl, bl)
  return out[:, :2]


# speedup vs baseline: 22.3204x; 1.1574x over previous
import functools

import jax
import jax.numpy as jnp
from jax import lax
from jax.experimental import pallas as pl
from jax.experimental.pallas import tpu as pltpu
from jax.experimental.pallas import tpu_sc as plsc

N = 50000
E = 800000
V = 10000
D = 64
G = 256

NC, NS = 2, 16
NP = 50176
EP = 819200
GP = 264
HD = D // 2

_mesh = plsc.VectorSubcoreMesh(
    core_axis_name="c", subcore_axis_name="s", num_cores=NC, num_subcores=NS)


def _loop(n, f):
  lax.fori_loop(0, n, lambda i, c: (f(i), 0)[1], 0)


def _f32(*shape):
  return jax.ShapeDtypeStruct(shape, jnp.float32)


@functools.partial(
    pl.kernel,
    out_type=[_f32(NP, D), _f32(NC, NP), _f32(NC, GP)],
    mesh=_mesh,
    compiler_params=pltpu.CompilerParams(use_tc_tiling_on_sc=False),
    scratch_types=[
        pltpu.VMEM((14, 112), jnp.int32),
        pltpu.VMEM((112, D), jnp.float32),
        pltpu.VMEM((8, 128), jnp.int32),
        pltpu.VMEM((14, 112), jnp.int32),
        pltpu.VMEM((128,), jnp.float32),
        pltpu.VMEM_SHARED((NP,), jnp.float32),
        pltpu.VMEM_SHARED((GP,), jnp.float32),
        pltpu.SemaphoreType.DMA,
    ],
)
def _sc_embed_hist(x2, dst2, batch2, embed, z1,
                   h0_out, deg_out, cnt_out,
                   xbuf, rows, dbuf, bbuf, ones, deg_sh, cnt_sh, sem):
  cid = lax.axis_index("c")
  sid = lax.axis_index("s")
  wid = sid * NC + cid

  for i in range(8):
    ones[pl.ds(i * 16, 16)] = jnp.ones((16,), jnp.float32)
  pltpu.sync_copy(z1, deg_sh.at[pl.ds(sid * 3136, 3136)])

  @pl.when(sid == 0)
  def _():
    pltpu.sync_copy(z1.at[pl.ds(0, GP)], cnt_sh)

  plsc.subcore_barrier()

  pltpu.sync_copy(x2.at[pl.ds(wid * 14, 14), :], xbuf)

  def emb_step(j):
    pltpu.async_copy(embed.at[xbuf.at[j]], rows, sem).wait()
    pltpu.sync_copy(rows, h0_out.at[pl.ds(wid * 1568 + j * 112, 112), :])
  _loop(14, emb_step)

  dbase = cid * 3200 + sid * 200

  def deg_step(blk):
    pltpu.sync_copy(dst2.at[pl.ds(dbase + blk * 8, 8), :], dbuf)
    for r in range(8):
      pltpu.sync_copy(ones, deg_sh.at[dbuf.at[r]], add=True)
  _loop(25, deg_step)

  pltpu.sync_copy(batch2.at[pl.ds(cid * 224 + sid * 14, 14), :], bbuf)

  def cnt_step(j):
    pltpu.sync_copy(ones.at[pl.ds(0, 112)], cnt_sh.at[bbuf.at[j]], add=True)
  _loop(14, cnt_step)

  plsc.subcore_barrier()
  pltpu.sync_copy(deg_sh.at[pl.ds(sid * 3136, 3136)],
                  deg_out.at[cid, pl.ds(sid * 3136, 3136)])

  @pl.when(sid == 0)
  def _():
    pltpu.sync_copy(cnt_sh, cnt_out.at[cid])


QD = HD // 2


@functools.partial(
    pl.kernel,
    out_type=_f32(2 * NP, HD),
    mesh=_mesh,
    compiler_params=pltpu.CompilerParams(use_tc_tiling_on_sc=False),
    scratch_types=[
        pltpu.VMEM((2, 20, 128), jnp.int32),
        pltpu.VMEM((2, 20, 128), jnp.int32),
        pltpu.VMEM((4, 128, QD), jnp.float32),
        pltpu.VMEM_SHARED((NP, QD), jnp.float32),
        pltpu.VMEM_SHARED((NP, QD), jnp.float32),
        pltpu.SemaphoreType.DMA,
        [pltpu.SemaphoreType.DMA] * 4,
        [pltpu.SemaphoreType.DMA] * 4,
    ],
)
def _sc_conv_scatter(u, src2, dst2, S_out,
                     sbuf, dbuf, rows, u_sp, S_sp, isem, gs, ss):
  cid = lax.axis_index("c")
  sid = lax.axis_index("s")

  def idx_fire(blk, par):
    base = sid * 400 + blk * 20
    pltpu.async_copy(src2.at[pl.ds(base, 20), :], sbuf.at[par], isem)
    pltpu.async_copy(dst2.at[pl.ds(base, 20), :], dbuf.at[par], isem)

  def idx_wait(blk, par):
    base = sid * 400 + blk * 20
    pltpu.make_async_copy(
        src2.at[pl.ds(base, 20), :], sbuf.at[par], isem).wait()
    pltpu.make_async_copy(
        dst2.at[pl.ds(base, 20), :], dbuf.at[par], isem).wait()

  def gfire(g, slot):
    p2 = lax.rem(g // 20, 2)
    r2 = lax.rem(g, 20)
    pltpu.async_copy(u_sp.at[sbuf.at[p2, r2]], rows.at[slot], gs[slot])

  def sdrain(slot):
    pltpu.make_async_copy(rows.at[slot], S_sp.at[dbuf.at[0, 0]],
                          ss[slot]).wait()

  for phase in range(2):
    nsl = pl.ds(sid * 3136, 3136)
    csl = pl.ds(phase * QD, QD)
    pltpu.async_copy(u.at[pl.ds(cid * NP + sid * 3136, 3136), csl],
                     u_sp.at[nsl, :], isem)
    pltpu.async_copy(u.at[pl.ds(cid * NP + sid * 3136, 3136), csl],
                     S_sp.at[nsl, :], gs[0])
    pltpu.make_async_copy(u.at[pl.ds(cid * NP + sid * 3136, 3136), csl],
                          u_sp.at[nsl, :], isem).wait()
    pltpu.make_async_copy(u.at[pl.ds(cid * NP + sid * 3136, 3136), csl],
                          S_sp.at[nsl, :], gs[0]).wait()
    plsc.subcore_barrier()

    idx_fire(0, 0)
    idx_wait(0, 0)
    gfire(0, 0)
    gfire(1, 1)

    def q_step(q):
      b = q // 5
      j = lax.rem(q, 5)
      p = lax.rem(b, 2)
      for c in range(4):
        g = 4 * q + c

        @pl.when(g >= 2)
        def _():
          sdrain((c - 2) % 4)

        @pl.when(g + 2 < 400)
        def _():
          gfire(g + 2, (c + 2) % 4)
        row = 4 * j + c
        pltpu.make_async_copy(u_sp.at[sbuf.at[p, row]], rows.at[c],
                              gs[c]).wait()
        pltpu.async_copy(rows.at[c], S_sp.at[dbuf.at[p, row]], ss[c],
                         add=True)

      @pl.when((j == 0) & (b + 1 < 20))
      def _():
        idx_fire(b + 1, 1 - p)

      @pl.when((j == 3) & (b + 1 < 20))
      def _():
        idx_wait(b + 1, 1 - p)
    _loop(100, q_step)

    sdrain(2)
    sdrain(3)

    plsc.subcore_barrier()
    pltpu.sync_copy(S_sp.at[nsl, :],
                    S_out.at[pl.ds(cid * NP + sid * 3136, 3136), csl])
    plsc.subcore_barrier()


@functools.partial(
    pl.kernel,
    out_type=_f32(NC, GP, D),
    mesh=_mesh,
    compiler_params=pltpu.CompilerParams(use_tc_tiling_on_sc=False),
    scratch_types=[
        pltpu.VMEM((14, 112), jnp.int32),
        pltpu.VMEM((112, D), jnp.float32),
        pltpu.VMEM_SHARED((GP, D), jnp.float32),
    ],
)
def _sc_pool(h2, batch2, zG, pool_out, bbuf, hbuf, pool_sh):
  cid = lax.axis_index("c")
  sid = lax.axis_index("s")

  @pl.when(sid == 0)
  def _():
    pltpu.sync_copy(zG, pool_sh)

  plsc.subcore_barrier()

  base = cid * 25088 + sid * 1568
  pltpu.sync_copy(batch2.at[pl.ds(cid * 224 + sid * 14, 14), :], bbuf)

  def pool_step(j):
    pltpu.sync_copy(h2.at[pl.ds(base + j * 112, 112), :], hbuf)
    pltpu.sync_copy(hbuf, pool_sh.at[bbuf.at[j]], add=True)
  _loop(14, pool_step)

  plsc.subcore_barrier()

  @pl.when(sid == 0)
  def _():
    pltpu.sync_copy(pool_sh, pool_out.at[cid])


RB = 1792
NJ = NP // RB


def _tc_u1_body(h0, deg, W1h, u1, dinv_out):
  dcol = lax.rsqrt(deg[...] + 1.0)
  dinv_out[...] = dcol
  u1[...] = jnp.dot(h0[...], W1h[0],
                    preferred_element_type=jnp.float32) * dcol


def _tc_u1(h0, deg_v, W1):
  return pl.pallas_call(
      _tc_u1_body,
      grid=(2 * NJ,),
      in_specs=[
          pl.BlockSpec((RB, D), lambda j: (lax.rem(j, NJ), 0)),
          pl.BlockSpec((RB, 1), lambda j: (lax.rem(j, NJ), 0)),
          pl.BlockSpec((1, D, HD), lambda j: (j // NJ, 0, 0)),
      ],
      out_specs=[
          pl.BlockSpec((RB, HD), lambda j: (j, 0)),
          pl.BlockSpec((RB, 1), lambda j: (lax.rem(j, NJ), 0)),
      ],
      out_shape=[_f32(2 * NP, HD), _f32(NP, 1)],
  )(h0, deg_v, W1)


def _tc_u2_body(Sa, Sb, dinv, b1, W2h, u2):
  dcol = dinv[...]
  ha = jnp.maximum(dcol * Sa[...] + b1[...][:, :HD], 0.0)
  hb = jnp.maximum(dcol * Sb[...] + b1[...][:, HD:], 0.0)
  h1 = jnp.concatenate([ha, hb], axis=1)
  u2[...] = jnp.dot(h1, W2h[0], preferred_element_type=jnp.float32) * dcol


def _tc_u2(S1, dinv, b1, W2):
  rmap = lambda j: (lax.rem(j, NJ), 0)
  rmapb = lambda j: (lax.rem(j, NJ) + NJ, 0)
  return pl.pallas_call(
      _tc_u2_body,
      grid=(2 * NJ,),
      in_specs=[
          pl.BlockSpec((RB, HD), rmap),
          pl.BlockSpec((RB, HD), rmapb),
          pl.BlockSpec((RB, 1), rmap),
          pl.BlockSpec((1, D), lambda j: (0, 0)),
          pl.BlockSpec((1, D, HD), lambda j: (j // NJ, 0, 0)),
      ],
      out_specs=pl.BlockSpec((RB, HD), lambda j: (j, 0)),
      out_shape=_f32(2 * NP, HD),
  )(S1, S1, dinv, b1, W2)


def _tc_h2_body(Sa, Sb, dinv, b2, h2):
  dcol = dinv[...]
  ha = jnp.maximum(dcol * Sa[...] + b2[...][:, :HD], 0.0)
  hb = jnp.maximum(dcol * Sb[...] + b2[...][:, HD:], 0.0)
  h2[...] = jnp.concatenate([ha, hb], axis=1)


def _tc_h2(S2, dinv, b2):
  rmap = lambda j: (j, 0)
  rmapb = lambda j: (j + NJ, 0)
  return pl.pallas_call(
      _tc_h2_body,
      grid=(NJ,),
      in_specs=[
          pl.BlockSpec((RB, HD), rmap),
          pl.BlockSpec((RB, HD), rmapb),
          pl.BlockSpec((RB, 1), rmap),
          pl.BlockSpec((1, D), lambda j: (0, 0)),
      ],
      out_specs=pl.BlockSpec((RB, D), rmap),
      out_shape=_f32(NP, D),
  )(S2, S2, dinv, b2)


def _tc_head_body(p0, p1, cnt, Wl, bl, out):
  g = (p0[...] + p1[...]) / jnp.maximum(cnt[...], 1.0)
  out[...] = jnp.dot(g, Wl[...], preferred_element_type=jnp.float32) + bl[...]


def _tc_head(p0, p1, cnt, Wl, bl):
  return pl.pallas_call(
      _tc_head_body,
      out_shape=_f32(G, 128),
  )(p0, p1, cnt, Wl, bl)


def kernel(x, edge_index, edge_type, batch, embed, W1, b1, W2, b2, Wlin, blin):
  del edge_type
  f32 = jnp.float32

  x2 = jnp.pad(x.astype(jnp.int32), (0, NP - N)).reshape(448, 112)
  src2 = jnp.pad(edge_index[0].astype(jnp.int32), (0, EP - E),
                 constant_values=N).reshape(6400, 128)
  dst2 = jnp.pad(edge_index[1].astype(jnp.int32), (0, EP - E),
                 constant_values=N).reshape(6400, 128)
  batch2 = jnp.pad(batch.astype(jnp.int32), (0, NP - N),
                   constant_values=G).reshape(448, 112)
  z1 = jnp.zeros((3136,), f32)
  zG = jnp.zeros((GP, D), f32)
  b1r = b1.reshape(1, D)
  b2r = b2.reshape(1, D)
  W1s = jnp.stack([W1[:, :HD], W1[:, HD:]])
  W2s = jnp.stack([W2[:, :HD], W2[:, HD:]])
  Wl = jnp.zeros((D, 128), f32).at[:, :2].set(Wlin)
  bl = jnp.zeros((1, 128), f32).at[0, :2].set(blin)

  h0, deg2, cnt2 = _sc_embed_hist(x2, dst2, batch2, embed, z1)
  deg_v = (deg2[0] + deg2[1]).reshape(NP, 1)
  cnt = (cnt2[0, :G] + cnt2[1, :G]).reshape(G, 1)

  u1, dinv = _tc_u1(h0, deg_v, W1s)
  S1 = _sc_conv_scatter(u1, src2, dst2)
  u2 = _tc_u2(S1, dinv, b1r, W2s)

  S2 = _sc_conv_scatter(u2, src2, dst2)
  h2 = _tc_h2(S2, dinv, b2r)

  pool = _sc_pool(h2, batch2, zG)
  out = _tc_head(pool[0, :G], pool[1, :G], cnt, Wl, bl)
  return out[:, :2]


# speedup vs baseline: 22.4094x; 1.0040x over previous
import functools

import jax
import jax.numpy as jnp
from jax import lax
from jax.experimental import pallas as pl
from jax.experimental.pallas import tpu as pltpu
from jax.experimental.pallas import tpu_sc as plsc

N = 50000
E = 800000
V = 10000
D = 64
G = 256

NC, NS = 2, 16
NP = 50176
EP = 819200
GP = 264
HD = D // 2

_mesh = plsc.VectorSubcoreMesh(
    core_axis_name="c", subcore_axis_name="s", num_cores=NC, num_subcores=NS)


def _loop(n, f):
  lax.fori_loop(0, n, lambda i, c: (f(i), 0)[1], 0)


def _f32(*shape):
  return jax.ShapeDtypeStruct(shape, jnp.float32)


@functools.partial(
    pl.kernel,
    out_type=[_f32(NP, D), _f32(NC, NP), _f32(NC, GP)],
    mesh=_mesh,
    compiler_params=pltpu.CompilerParams(use_tc_tiling_on_sc=False),
    scratch_types=[
        pltpu.VMEM((14, 112), jnp.int32),
        pltpu.VMEM((112, D), jnp.float32),
        pltpu.VMEM((8, 128), jnp.int32),
        pltpu.VMEM((14, 112), jnp.int32),
        pltpu.VMEM((128,), jnp.float32),
        pltpu.VMEM_SHARED((NP,), jnp.float32),
        pltpu.VMEM_SHARED((GP,), jnp.float32),
        pltpu.SemaphoreType.DMA,
    ],
)
def _sc_embed_hist(x2, dst2, batch2, embed, z1,
                   h0_out, deg_out, cnt_out,
                   xbuf, rows, dbuf, bbuf, ones, deg_sh, cnt_sh, sem):
  cid = lax.axis_index("c")
  sid = lax.axis_index("s")
  wid = sid * NC + cid

  for i in range(8):
    ones[pl.ds(i * 16, 16)] = jnp.ones((16,), jnp.float32)
  pltpu.sync_copy(z1, deg_sh.at[pl.ds(sid * 3136, 3136)])

  @pl.when(sid == 0)
  def _():
    pltpu.sync_copy(z1.at[pl.ds(0, GP)], cnt_sh)

  plsc.subcore_barrier()

  pltpu.sync_copy(x2.at[pl.ds(wid * 14, 14), :], xbuf)

  def emb_step(j):
    pltpu.async_copy(embed.at[xbuf.at[j]], rows, sem).wait()
    pltpu.sync_copy(rows, h0_out.at[pl.ds(wid * 1568 + j * 112, 112), :])
  _loop(14, emb_step)

  dbase = cid * 3200 + sid * 200

  def deg_step(blk):
    pltpu.sync_copy(dst2.at[pl.ds(dbase + blk * 8, 8), :], dbuf)
    for r in range(8):
      pltpu.sync_copy(ones, deg_sh.at[dbuf.at[r]], add=True)
  _loop(25, deg_step)

  pltpu.sync_copy(batch2.at[pl.ds(cid * 224 + sid * 14, 14), :], bbuf)

  def cnt_step(j):
    pltpu.sync_copy(ones.at[pl.ds(0, 112)], cnt_sh.at[bbuf.at[j]], add=True)
  _loop(14, cnt_step)

  plsc.subcore_barrier()
  pltpu.sync_copy(deg_sh.at[pl.ds(sid * 3136, 3136)],
                  deg_out.at[cid, pl.ds(sid * 3136, 3136)])

  @pl.when(sid == 0)
  def _():
    pltpu.sync_copy(cnt_sh, cnt_out.at[cid])


QD = HD // 2


@functools.partial(
    pl.kernel,
    out_type=_f32(2 * NP, HD),
    mesh=_mesh,
    compiler_params=pltpu.CompilerParams(use_tc_tiling_on_sc=False),
    scratch_types=[
        pltpu.VMEM((3, 8, 128), jnp.int32),
        pltpu.VMEM((3, 8, 128), jnp.int32),
        pltpu.VMEM((8, 128, QD), jnp.float32),
        pltpu.VMEM_SHARED((NP, QD), jnp.float32),
        pltpu.VMEM_SHARED((NP, QD), jnp.float32),
        pltpu.SemaphoreType.DMA,
        [pltpu.SemaphoreType.DMA] * 8,
        [pltpu.SemaphoreType.DMA] * 8,
    ],
)
def _sc_conv_scatter(u, src2, dst2, S_out,
                     sbuf, dbuf, rows, u_sp, S_sp, isem, gs, ss):
  cid = lax.axis_index("c")
  sid = lax.axis_index("s")

  def idx_fire(blk, par):
    base = sid * 400 + blk * 8
    pltpu.async_copy(src2.at[pl.ds(base, 8), :], sbuf.at[par], isem)
    pltpu.async_copy(dst2.at[pl.ds(base, 8), :], dbuf.at[par], isem)

  def idx_wait(blk, par):
    base = sid * 400 + blk * 8
    pltpu.make_async_copy(
        src2.at[pl.ds(base, 8), :], sbuf.at[par], isem).wait()
    pltpu.make_async_copy(
        dst2.at[pl.ds(base, 8), :], dbuf.at[par], isem).wait()

  def gfire(g, slot):
    p2 = lax.rem(g // 8, 3)
    r2 = lax.rem(g, 8)
    pltpu.async_copy(u_sp.at[sbuf.at[p2, r2]], rows.at[slot], gs[slot])

  def sdrain(slot):
    pltpu.make_async_copy(rows.at[slot], S_sp.at[dbuf.at[0, 0]],
                          ss[slot]).wait()

  for phase in range(2):
    nsl = pl.ds(sid * 3136, 3136)
    csl = pl.ds(phase * QD, QD)
    pltpu.async_copy(u.at[pl.ds(cid * NP + sid * 3136, 3136), csl],
                     u_sp.at[nsl, :], isem)
    pltpu.async_copy(u.at[pl.ds(cid * NP + sid * 3136, 3136), csl],
                     S_sp.at[nsl, :], gs[0])
    pltpu.make_async_copy(u.at[pl.ds(cid * NP + sid * 3136, 3136), csl],
                          u_sp.at[nsl, :], isem).wait()
    pltpu.make_async_copy(u.at[pl.ds(cid * NP + sid * 3136, 3136), csl],
                          S_sp.at[nsl, :], gs[0]).wait()
    plsc.subcore_barrier()

    idx_fire(0, 0)
    idx_fire(1, 1)
    idx_wait(0, 0)
    for c in range(4):
      gfire(c, c)

    def q_step(q):
      p = lax.rem(q, 3)
      for c in range(8):
        g = 8 * q + c

        @pl.when(g >= 4)
        def _():
          sdrain((c - 4) % 8)
        if c == 4:
          @pl.when(q + 1 < 50)
          def _():
            idx_wait(q + 1, lax.rem(q + 1, 3))

          @pl.when(q + 2 < 50)
          def _():
            idx_fire(q + 2, lax.rem(q + 2, 3))

        @pl.when(g + 4 < 400)
        def _():
          gfire(g + 4, (c + 4) % 8)
        pltpu.make_async_copy(u_sp.at[sbuf.at[p, c]], rows.at[c],
                              gs[c]).wait()
        pltpu.async_copy(rows.at[c], S_sp.at[dbuf.at[p, c]], ss[c],
                         add=True)
    _loop(50, q_step)

    for c in range(4, 8):
      sdrain(c)

    plsc.subcore_barrier()
    pltpu.sync_copy(S_sp.at[nsl, :],
                    S_out.at[pl.ds(cid * NP + sid * 3136, 3136), csl])
    plsc.subcore_barrier()


@functools.partial(
    pl.kernel,
    out_type=_f32(NC, GP, D),
    mesh=_mesh,
    compiler_params=pltpu.CompilerParams(use_tc_tiling_on_sc=False),
    scratch_types=[
        pltpu.VMEM((14, 112), jnp.int32),
        pltpu.VMEM((112, D), jnp.float32),
        pltpu.VMEM_SHARED((GP, D), jnp.float32),
    ],
)
def _sc_pool(h2, batch2, zG, pool_out, bbuf, hbuf, pool_sh):
  cid = lax.axis_index("c")
  sid = lax.axis_index("s")

  @pl.when(sid == 0)
  def _():
    pltpu.sync_copy(zG, pool_sh)

  plsc.subcore_barrier()

  base = cid * 25088 + sid * 1568
  pltpu.sync_copy(batch2.at[pl.ds(cid * 224 + sid * 14, 14), :], bbuf)

  def pool_step(j):
    pltpu.sync_copy(h2.at[pl.ds(base + j * 112, 112), :], hbuf)
    pltpu.sync_copy(hbuf, pool_sh.at[bbuf.at[j]], add=True)
  _loop(14, pool_step)

  plsc.subcore_barrier()

  @pl.when(sid == 0)
  def _():
    pltpu.sync_copy(pool_sh, pool_out.at[cid])


RB = 1792
NJ = NP // RB


def _tc_u1_body(h0, deg, W1h, u1, dinv_out):
  dcol = lax.rsqrt(deg[...] + 1.0)
  dinv_out[...] = dcol
  u1[...] = jnp.dot(h0[...], W1h[0],
                    preferred_element_type=jnp.float32) * dcol


def _tc_u1(h0, deg_v, W1):
  return pl.pallas_call(
      _tc_u1_body,
      grid=(2 * NJ,),
      in_specs=[
          pl.BlockSpec((RB, D), lambda j: (lax.rem(j, NJ), 0)),
          pl.BlockSpec((RB, 1), lambda j: (lax.rem(j, NJ), 0)),
          pl.BlockSpec((1, D, HD), lambda j: (j // NJ, 0, 0)),
      ],
      out_specs=[
          pl.BlockSpec((RB, HD), lambda j: (j, 0)),
          pl.BlockSpec((RB, 1), lambda j: (lax.rem(j, NJ), 0)),
      ],
      out_shape=[_f32(2 * NP, HD), _f32(NP, 1)],
  )(h0, deg_v, W1)


def _tc_u2_body(Sa, Sb, dinv, b1, W2h, u2):
  dcol = dinv[...]
  ha = jnp.maximum(dcol * Sa[...] + b1[...][:, :HD], 0.0)
  hb = jnp.maximum(dcol * Sb[...] + b1[...][:, HD:], 0.0)
  h1 = jnp.concatenate([ha, hb], axis=1)
  u2[...] = jnp.dot(h1, W2h[0], preferred_element_type=jnp.float32) * dcol


def _tc_u2(S1, dinv, b1, W2):
  rmap = lambda j: (lax.rem(j, NJ), 0)
  rmapb = lambda j: (lax.rem(j, NJ) + NJ, 0)
  return pl.pallas_call(
      _tc_u2_body,
      grid=(2 * NJ,),
      in_specs=[
          pl.BlockSpec((RB, HD), rmap),
          pl.BlockSpec((RB, HD), rmapb),
          pl.BlockSpec((RB, 1), rmap),
          pl.BlockSpec((1, D), lambda j: (0, 0)),
          pl.BlockSpec((1, D, HD), lambda j: (j // NJ, 0, 0)),
      ],
      out_specs=pl.BlockSpec((RB, HD), lambda j: (j, 0)),
      out_shape=_f32(2 * NP, HD),
  )(S1, S1, dinv, b1, W2)


def _tc_h2_body(Sa, Sb, dinv, b2, h2):
  dcol = dinv[...]
  ha = jnp.maximum(dcol * Sa[...] + b2[...][:, :HD], 0.0)
  hb = jnp.maximum(dcol * Sb[...] + b2[...][:, HD:], 0.0)
  h2[...] = jnp.concatenate([ha, hb], axis=1)


def _tc_h2(S2, dinv, b2):
  rmap = lambda j: (j, 0)
  rmapb = lambda j: (j + NJ, 0)
  return pl.pallas_call(
      _tc_h2_body,
      grid=(NJ,),
      in_specs=[
          pl.BlockSpec((RB, HD), rmap),
          pl.BlockSpec((RB, HD), rmapb),
          pl.BlockSpec((RB, 1), rmap),
          pl.BlockSpec((1, D), lambda j: (0, 0)),
      ],
      out_specs=pl.BlockSpec((RB, D), rmap),
      out_shape=_f32(NP, D),
  )(S2, S2, dinv, b2)


def _tc_head_body(p0, p1, cnt, Wl, bl, out):
  g = (p0[...] + p1[...]) / jnp.maximum(cnt[...], 1.0)
  out[...] = jnp.dot(g, Wl[...], preferred_element_type=jnp.float32) + bl[...]


def _tc_head(p0, p1, cnt, Wl, bl):
  return pl.pallas_call(
      _tc_head_body,
      out_shape=_f32(G, 128),
  )(p0, p1, cnt, Wl, bl)


def kernel(x, edge_index, edge_type, batch, embed, W1, b1, W2, b2, Wlin, blin):
  del edge_type
  f32 = jnp.float32

  x2 = jnp.pad(x.astype(jnp.int32), (0, NP - N)).reshape(448, 112)
  src2 = jnp.pad(edge_index[0].astype(jnp.int32), (0, EP - E),
                 constant_values=N).reshape(6400, 128)
  dst2 = jnp.pad(edge_index[1].astype(jnp.int32), (0, EP - E),
                 constant_values=N).reshape(6400, 128)
  batch2 = jnp.pad(batch.astype(jnp.int32), (0, NP - N),
                   constant_values=G).reshape(448, 112)
  z1 = jnp.zeros((3136,), f32)
  zG = jnp.zeros((GP, D), f32)
  b1r = b1.reshape(1, D)
  b2r = b2.reshape(1, D)
  W1s = jnp.stack([W1[:, :HD], W1[:, HD:]])
  W2s = jnp.stack([W2[:, :HD], W2[:, HD:]])
  Wl = jnp.zeros((D, 128), f32).at[:, :2].set(Wlin)
  bl = jnp.zeros((1, 128), f32).at[0, :2].set(blin)

  h0, deg2, cnt2 = _sc_embed_hist(x2, dst2, batch2, embed, z1)
  deg_v = (deg2[0] + deg2[1]).reshape(NP, 1)
  cnt = (cnt2[0, :G] + cnt2[1, :G]).reshape(G, 1)

  u1, dinv = _tc_u1(h0, deg_v, W1s)
  S1 = _sc_conv_scatter(u1, src2, dst2)
  u2 = _tc_u2(S1, dinv, b1r, W2s)

  S2 = _sc_conv_scatter(u2, src2, dst2)
  h2 = _tc_h2(S2, dinv, b2r)

  pool = _sc_pool(h2, batch2, zG)
  out = _tc_head(pool[0, :G], pool[1, :G], cnt, Wl, bl)
  return out[:, :2]


# speedup vs baseline: 22.5847x; 1.0078x over previous
import functools

import jax
import jax.numpy as jnp
from jax import lax
from jax.experimental import pallas as pl
from jax.experimental.pallas import tpu as pltpu
from jax.experimental.pallas import tpu_sc as plsc

N = 50000
E = 800000
V = 10000
D = 64
G = 256

NC, NS = 2, 16
NP = 50176
EP = 819200
GP = 264
HD = D // 2

_mesh = plsc.VectorSubcoreMesh(
    core_axis_name="c", subcore_axis_name="s", num_cores=NC, num_subcores=NS)


def _loop(n, f):
  lax.fori_loop(0, n, lambda i, c: (f(i), 0)[1], 0)


def _f32(*shape):
  return jax.ShapeDtypeStruct(shape, jnp.float32)


@functools.partial(
    pl.kernel,
    out_type=[_f32(NP, D), _f32(NC, NP), _f32(NC, GP)],
    mesh=_mesh,
    compiler_params=pltpu.CompilerParams(use_tc_tiling_on_sc=False),
    scratch_types=[
        pltpu.VMEM((14, 112), jnp.int32),
        pltpu.VMEM((2, 112, D), jnp.float32),
        pltpu.VMEM((2, 8, 128), jnp.int32),
        pltpu.VMEM((14, 112), jnp.int32),
        pltpu.VMEM((128,), jnp.float32),
        pltpu.VMEM_SHARED((NP,), jnp.float32),
        pltpu.VMEM_SHARED((GP,), jnp.float32),
        pltpu.SemaphoreType.DMA,
        [pltpu.SemaphoreType.DMA] * 2,
        [pltpu.SemaphoreType.DMA] * 2,
        [pltpu.SemaphoreType.DMA] * 8,
    ],
)
def _sc_embed_hist(x2, dst2, batch2, embed, z1,
                   h0_out, deg_out, cnt_out,
                   xbuf, rows, dbuf, bbuf, ones, deg_sh, cnt_sh, sem,
                   es, os_, ds):
  cid = lax.axis_index("c")
  sid = lax.axis_index("s")
  wid = sid * NC + cid

  for i in range(8):
    ones[pl.ds(i * 16, 16)] = jnp.ones((16,), jnp.float32)
  pltpu.sync_copy(z1, deg_sh.at[pl.ds(sid * 3136, 3136)])

  @pl.when(sid == 0)
  def _():
    pltpu.sync_copy(z1.at[pl.ds(0, GP)], cnt_sh)

  plsc.subcore_barrier()

  pltpu.sync_copy(x2.at[pl.ds(wid * 14, 14), :], xbuf)

  def egfire(j, k):
    pltpu.async_copy(embed.at[xbuf.at[j]], rows.at[k], es[k])

  def eofire(j, k):
    pltpu.async_copy(rows.at[k], h0_out.at[pl.ds(wid * 1568 + j * 112, 112),
                                           :], os_[k])

  def eodrain(j, k):
    pltpu.make_async_copy(rows.at[k], h0_out.at[pl.ds(wid * 1568 + j * 112,
                                                      112), :], os_[k]).wait()

  egfire(0, 0)

  def emb_step(t):
    for k in range(2):
      j = 2 * t + k

      @pl.when(j >= 1)
      def _():
        eodrain(j - 1, 1 - k)

      @pl.when(j + 1 < 14)
      def _():
        egfire(j + 1, 1 - k)
      pltpu.make_async_copy(embed.at[xbuf.at[j]], rows.at[k], es[k]).wait()
      eofire(j, k)
  _loop(7, emb_step)
  eodrain(13, 1)

  dbase = cid * 3200 + sid * 200

  def deg_step(blk):
    p = lax.rem(blk, 2)

    @pl.when(blk > 0)
    def _():
      for r in range(8):
        pltpu.make_async_copy(ones, deg_sh.at[dbuf.at[0, 0]], ds[r]).wait()
    pltpu.sync_copy(dst2.at[pl.ds(dbase + blk * 8, 8), :], dbuf.at[p])
    for r in range(8):
      pltpu.async_copy(ones, deg_sh.at[dbuf.at[p, r]], ds[r], add=True)
  _loop(25, deg_step)
  for r in range(8):
    pltpu.make_async_copy(ones, deg_sh.at[dbuf.at[0, 0]], ds[r]).wait()

  pltpu.sync_copy(batch2.at[pl.ds(cid * 224 + sid * 14, 14), :], bbuf)

  def cnt_step(j):
    pltpu.sync_copy(ones.at[pl.ds(0, 112)], cnt_sh.at[bbuf.at[j]], add=True)
  _loop(14, cnt_step)

  plsc.subcore_barrier()
  pltpu.sync_copy(deg_sh.at[pl.ds(sid * 3136, 3136)],
                  deg_out.at[cid, pl.ds(sid * 3136, 3136)])

  @pl.when(sid == 0)
  def _():
    pltpu.sync_copy(cnt_sh, cnt_out.at[cid])


QD = HD // 2


@functools.partial(
    pl.kernel,
    out_type=_f32(2 * NP, HD),
    mesh=_mesh,
    compiler_params=pltpu.CompilerParams(use_tc_tiling_on_sc=False),
    scratch_types=[
        pltpu.VMEM((3, 8, 128), jnp.int32),
        pltpu.VMEM((3, 8, 128), jnp.int32),
        pltpu.VMEM((8, 128, QD), jnp.float32),
        pltpu.VMEM_SHARED((NP, QD), jnp.float32),
        pltpu.VMEM_SHARED((NP, QD), jnp.float32),
        pltpu.SemaphoreType.DMA,
        [pltpu.SemaphoreType.DMA] * 8,
        [pltpu.SemaphoreType.DMA] * 8,
    ],
)
def _sc_conv_scatter(u, src2, dst2, S_out,
                     sbuf, dbuf, rows, u_sp, S_sp, isem, gs, ss):
  cid = lax.axis_index("c")
  sid = lax.axis_index("s")

  def idx_fire(blk, par):
    base = sid * 400 + blk * 8
    pltpu.async_copy(src2.at[pl.ds(base, 8), :], sbuf.at[par], isem)
    pltpu.async_copy(dst2.at[pl.ds(base, 8), :], dbuf.at[par], isem)

  def idx_wait(blk, par):
    base = sid * 400 + blk * 8
    pltpu.make_async_copy(
        src2.at[pl.ds(base, 8), :], sbuf.at[par], isem).wait()
    pltpu.make_async_copy(
        dst2.at[pl.ds(base, 8), :], dbuf.at[par], isem).wait()

  def gfire(g, slot):
    p2 = lax.rem(g // 8, 3)
    r2 = lax.rem(g, 8)
    pltpu.async_copy(u_sp.at[sbuf.at[p2, r2]], rows.at[slot], gs[slot])

  def sdrain(slot):
    pltpu.make_async_copy(rows.at[slot], S_sp.at[dbuf.at[0, 0]],
                          ss[slot]).wait()

  for phase in range(2):
    nsl = pl.ds(sid * 3136, 3136)
    csl = pl.ds(phase * QD, QD)
    pltpu.async_copy(u.at[pl.ds(cid * NP + sid * 3136, 3136), csl],
                     u_sp.at[nsl, :], isem)
    pltpu.async_copy(u.at[pl.ds(cid * NP + sid * 3136, 3136), csl],
                     S_sp.at[nsl, :], gs[0])
    pltpu.make_async_copy(u.at[pl.ds(cid * NP + sid * 3136, 3136), csl],
                          u_sp.at[nsl, :], isem).wait()
    pltpu.make_async_copy(u.at[pl.ds(cid * NP + sid * 3136, 3136), csl],
                          S_sp.at[nsl, :], gs[0]).wait()
    plsc.subcore_barrier()

    idx_fire(0, 0)
    idx_fire(1, 1)
    idx_wait(0, 0)
    for c in range(4):
      gfire(c, c)

    def q_step(q):
      p = lax.rem(q, 3)
      for c in range(8):
        g = 8 * q + c

        @pl.when(g >= 4)
        def _():
          sdrain((c - 4) % 8)
        if c == 4:
          @pl.when(q + 1 < 50)
          def _():
            idx_wait(q + 1, lax.rem(q + 1, 3))

          @pl.when(q + 2 < 50)
          def _():
            idx_fire(q + 2, lax.rem(q + 2, 3))

        @pl.when(g + 4 < 400)
        def _():
          gfire(g + 4, (c + 4) % 8)
        pltpu.make_async_copy(u_sp.at[sbuf.at[p, c]], rows.at[c],
                              gs[c]).wait()
        pltpu.async_copy(rows.at[c], S_sp.at[dbuf.at[p, c]], ss[c],
                         add=True)
    _loop(50, q_step)

    for c in range(4, 8):
      sdrain(c)

    plsc.subcore_barrier()
    pltpu.sync_copy(S_sp.at[nsl, :],
                    S_out.at[pl.ds(cid * NP + sid * 3136, 3136), csl])
    plsc.subcore_barrier()


@functools.partial(
    pl.kernel,
    out_type=_f32(NC, GP, D),
    mesh=_mesh,
    compiler_params=pltpu.CompilerParams(use_tc_tiling_on_sc=False),
    scratch_types=[
        pltpu.VMEM((14, 112), jnp.int32),
        pltpu.VMEM((112, D), jnp.float32),
        pltpu.VMEM_SHARED((GP, D), jnp.float32),
    ],
)
def _sc_pool(h2, batch2, zG, pool_out, bbuf, hbuf, pool_sh):
  cid = lax.axis_index("c")
  sid = lax.axis_index("s")

  @pl.when(sid == 0)
  def _():
    pltpu.sync_copy(zG, pool_sh)

  plsc.subcore_barrier()

  base = cid * 25088 + sid * 1568
  pltpu.sync_copy(batch2.at[pl.ds(cid * 224 + sid * 14, 14), :], bbuf)

  def pool_step(j):
    pltpu.sync_copy(h2.at[pl.ds(base + j * 112, 112), :], hbuf)
    pltpu.sync_copy(hbuf, pool_sh.at[bbuf.at[j]], add=True)
  _loop(14, pool_step)

  plsc.subcore_barrier()

  @pl.when(sid == 0)
  def _():
    pltpu.sync_copy(pool_sh, pool_out.at[cid])


RB = 1792
NJ = NP // RB


def _tc_u1_body(h0, deg, W1h, u1, dinv_out):
  dcol = lax.rsqrt(deg[...] + 1.0)
  dinv_out[...] = dcol
  u1[...] = jnp.dot(h0[...], W1h[0],
                    preferred_element_type=jnp.float32) * dcol


def _tc_u1(h0, deg_v, W1):
  return pl.pallas_call(
      _tc_u1_body,
      grid=(2 * NJ,),
      in_specs=[
          pl.BlockSpec((RB, D), lambda j: (lax.rem(j, NJ), 0)),
          pl.BlockSpec((RB, 1), lambda j: (lax.rem(j, NJ), 0)),
          pl.BlockSpec((1, D, HD), lambda j: (j // NJ, 0, 0)),
      ],
      out_specs=[
          pl.BlockSpec((RB, HD), lambda j: (j, 0)),
          pl.BlockSpec((RB, 1), lambda j: (lax.rem(j, NJ), 0)),
      ],
      out_shape=[_f32(2 * NP, HD), _f32(NP, 1)],
  )(h0, deg_v, W1)


def _tc_u2_body(Sa, Sb, dinv, b1, W2h, u2):
  dcol = dinv[...]
  ha = jnp.maximum(dcol * Sa[...] + b1[...][:, :HD], 0.0)
  hb = jnp.maximum(dcol * Sb[...] + b1[...][:, HD:], 0.0)
  h1 = jnp.concatenate([ha, hb], axis=1)
  u2[...] = jnp.dot(h1, W2h[0], preferred_element_type=jnp.float32) * dcol


def _tc_u2(S1, dinv, b1, W2):
  rmap = lambda j: (lax.rem(j, NJ), 0)
  rmapb = lambda j: (lax.rem(j, NJ) + NJ, 0)
  return pl.pallas_call(
      _tc_u2_body,
      grid=(2 * NJ,),
      in_specs=[
          pl.BlockSpec((RB, HD), rmap),
          pl.BlockSpec((RB, HD), rmapb),
          pl.BlockSpec((RB, 1), rmap),
          pl.BlockSpec((1, D), lambda j: (0, 0)),
          pl.BlockSpec((1, D, HD), lambda j: (j // NJ, 0, 0)),
      ],
      out_specs=pl.BlockSpec((RB, HD), lambda j: (j, 0)),
      out_shape=_f32(2 * NP, HD),
  )(S1, S1, dinv, b1, W2)


def _tc_h2_body(Sa, Sb, dinv, b2, h2):
  dcol = dinv[...]
  ha = jnp.maximum(dcol * Sa[...] + b2[...][:, :HD], 0.0)
  hb = jnp.maximum(dcol * Sb[...] + b2[...][:, HD:], 0.0)
  h2[...] = jnp.concatenate([ha, hb], axis=1)


def _tc_h2(S2, dinv, b2):
  rmap = lambda j: (j, 0)
  rmapb = lambda j: (j + NJ, 0)
  return pl.pallas_call(
      _tc_h2_body,
      grid=(NJ,),
      in_specs=[
          pl.BlockSpec((RB, HD), rmap),
          pl.BlockSpec((RB, HD), rmapb),
          pl.BlockSpec((RB, 1), rmap),
          pl.BlockSpec((1, D), lambda j: (0, 0)),
      ],
      out_specs=pl.BlockSpec((RB, D), rmap),
      out_shape=_f32(NP, D),
  )(S2, S2, dinv, b2)


def _tc_head_body(p0, p1, cnt, Wl, bl, out):
  g = (p0[...] + p1[...]) / jnp.maximum(cnt[...], 1.0)
  out[...] = jnp.dot(g, Wl[...], preferred_element_type=jnp.float32) + bl[...]


def _tc_head(p0, p1, cnt, Wl, bl):
  return pl.pallas_call(
      _tc_head_body,
      out_shape=_f32(G, 128),
  )(p0, p1, cnt, Wl, bl)


def kernel(x, edge_index, edge_type, batch, embed, W1, b1, W2, b2, Wlin, blin):
  del edge_type
  f32 = jnp.float32

  x2 = jnp.pad(x.astype(jnp.int32), (0, NP - N)).reshape(448, 112)
  src2 = jnp.pad(edge_index[0].astype(jnp.int32), (0, EP - E),
                 constant_values=N).reshape(6400, 128)
  dst2 = jnp.pad(edge_index[1].astype(jnp.int32), (0, EP - E),
                 constant_values=N).reshape(6400, 128)
  batch2 = jnp.pad(batch.astype(jnp.int32), (0, NP - N),
                   constant_values=G).reshape(448, 112)
  z1 = jnp.zeros((3136,), f32)
  zG = jnp.zeros((GP, D), f32)
  b1r = b1.reshape(1, D)
  b2r = b2.reshape(1, D)
  W1s = jnp.stack([W1[:, :HD], W1[:, HD:]])
  W2s = jnp.stack([W2[:, :HD], W2[:, HD:]])
  Wl = jnp.zeros((D, 128), f32).at[:, :2].set(Wlin)
  bl = jnp.zeros((1, 128), f32).at[0, :2].set(blin)

  h0, deg2, cnt2 = _sc_embed_hist(x2, dst2, batch2, embed, z1)
  deg_v = (deg2[0] + deg2[1]).reshape(NP, 1)
  cnt = (cnt2[0, :G] + cnt2[1, :G]).reshape(G, 1)

  u1, dinv = _tc_u1(h0, deg_v, W1s)
  S1 = _sc_conv_scatter(u1, src2, dst2)
  u2 = _tc_u2(S1, dinv, b1r, W2s)

  S2 = _sc_conv_scatter(u2, src2, dst2)
  h2 = _tc_h2(S2, dinv, b2r)

  pool = _sc_pool(h2, batch2, zG)
  out = _tc_head(pool[0, :G], pool[1, :G], cnt, Wl, bl)
  return out[:, :2]
